# Initial kernel scaffold; baseline (speedup 1.0000x reference)
import jax
import jax.numpy as jnp
from jax.experimental import pallas as pl


def kernel(self_pos, teammate_positions, opponent_positions, opponent_last_known_positions, self_feat, fwd_lidar, rear_lidar, teammates, opponents, opponents_last_known, opponent_masks, agent_map, unmasked_agent_map, table, fW1, fb1, fW2, fb2, fW3, fb3, fln_s, fln_b, rW1, rb1, rW2, rb2, rW3, rb3, rln_s, rln_b, train):
    raise NotImplementedError("write your pallas kernel here")


# speedup vs baseline: 1.0000x; 1.0000x over previous
import functools

import jax
import jax.numpy as jnp
import numpy as np
from jax import lax
from jax.experimental import pallas as pl
from jax.experimental.pallas import tpu as pltpu

L_LEVELS = 16
T_SIZE = 2 ** 14
N_FEAT = 2
N_MIN = 16
N_MAX = 1024
_B_GROWTH = np.exp((np.log(N_MAX) - np.log(N_MIN)) / (L_LEVELS - 1))
_P0 = np.uint32(1)
_P1 = np.uint32(2654435761)


def _hash_encode(pos, table, scale=1.0):
    pos = pos * scale
    outs = []
    for l in range(L_LEVELS):
        Nl = int(np.floor(N_MIN * (_B_GROWTH ** l)))
        x = pos * Nl
        x0 = jnp.floor(x)
        frac = x - x0
        xi = x0.astype(jnp.int32)
        feats = jnp.zeros((pos.shape[0], N_FEAT), dtype=table.dtype)
        for dx in (0, 1):
            for dy in (0, 1):
                corner = xi + jnp.array([dx, dy], dtype=jnp.int32)
                if (Nl + 1) ** 2 <= T_SIZE:
                    idx = corner[:, 0] * (Nl + 1) + corner[:, 1]
                else:
                    c = corner.astype(jnp.uint32)
                    h = (c[:, 0] * _P0) ^ (c[:, 1] * _P1)
                    idx = (h & jnp.uint32(T_SIZE - 1)).astype(jnp.int32)
                wx = frac[:, 0] if dx == 1 else 1.0 - frac[:, 0]
                wy = frac[:, 1] if dy == 1 else 1.0 - frac[:, 1]
                feats = feats + (wx * wy)[:, None] * table[l][idx]
        outs.append(feats)
    return jnp.concatenate(outs, axis=-1)


def _conv1d(x, W, b):
    y = lax.conv_general_dilated(x, W, window_strides=(2,), padding='SAME',
                                 dimension_numbers=('NWC', 'WIO', 'NWC'))
    return y + b


def _lidar_enc(lidar, W1, b1, W2, b2, W3, b3, ln_s, ln_b):
    lidar = jnp.swapaxes(lidar, -2, -3)
    lidar = lidar.reshape(*lidar.shape[0:-2], -1)
    x = jax.nn.leaky_relu(_conv1d(lidar, W1, b1))
    x = jax.nn.leaky_relu(_conv1d(x, W2, b2))
    x = _conv1d(x, W3, b3)
    x = x.reshape(*x.shape[:-2], -1)
    mu = jnp.mean(x, axis=-1, keepdims=True)
    var = jnp.var(x, axis=-1, keepdims=True)
    x = (x - mu) / jnp.sqrt(var + 1e-6) * ln_s + ln_b
    return jax.nn.leaky_relu(x)


def kernel(self_pos, teammate_positions, opponent_positions,
           opponent_last_known_positions, self_feat, fwd_lidar, rear_lidar,
           teammates, opponents, opponents_last_known, opponent_masks,
           agent_map, unmasked_agent_map, table, fW1, fb1, fW2, fb2, fW3, fb3,
           fln_s, fln_b, rW1, rb1, rW2, rb2, rW3, rb3, rln_s, rln_b, train):
    B = self_pos.shape[0]
    enc_self = _hash_encode(self_pos, table, 1.0)
    tp_shape = teammate_positions.shape
    op_shape = opponent_positions.shape
    ol_shape = opponent_last_known_positions.shape
    enc_tm = _hash_encode(teammate_positions.reshape(-1, tp_shape[-1]), table).reshape(*tp_shape[:-1], -1)
    enc_op = _hash_encode(opponent_positions.reshape(-1, op_shape[-1]), table).reshape(*op_shape[:-1], -1)
    enc_ol = _hash_encode(opponent_last_known_positions.reshape(-1, ol_shape[-1]), table).reshape(*ol_shape[:-1], -1)
    fwd_l = _lidar_enc(fwd_lidar, fW1, fb1, fW2, fb2, fW3, fb3, fln_s, fln_b)
    rear_l = _lidar_enc(rear_lidar, rW1, rb1, rW2, rb2, rW3, rb3, rln_s, rln_b)
    self_ob = jnp.concatenate([enc_self, self_feat, fwd_l, rear_l], axis=-1)
    tm = jnp.concatenate([enc_tm, teammates], axis=-1)
    op = jnp.concatenate([enc_op, opponents], axis=-1)
    ol = jnp.concatenate([enc_ol, opponents_last_known], axis=-1)
    return (self_ob, tm, op, ol, opponent_masks, agent_map, unmasked_agent_map)


# speedup vs baseline: 65.3544x; 65.3520x over previous
import functools

import jax
import jax.numpy as jnp
import numpy as np
from jax import lax
from jax.experimental import pallas as pl
from jax.experimental.pallas import tpu as pltpu
from jax.experimental.pallas import tpu_sc as plsc

L_LEVELS = 16
T_SIZE = 2 ** 14
N_FEAT = 2
N_MIN = 16
N_MAX = 1024
_B_GROWTH = np.exp((np.log(N_MAX) - np.log(N_MIN)) / (L_LEVELS - 1))
_P1 = np.uint32(2654435761)

_NLS = [int(np.floor(N_MIN * (_B_GROWTH ** l))) for l in range(L_LEVELS)]

_NC = 2
_NS = 16
_NW = _NC * _NS
_VL = 16


def _sc_hash_encode(t0, t1, px, py):
    Q = px.shape[0]
    assert Q % (_NW * _VL) == 0
    chunk = Q // _NW
    n_grp = chunk // _VL
    mesh = plsc.VectorSubcoreMesh(core_axis_name="c", subcore_axis_name="s")

    @functools.partial(
        pl.kernel,
        out_type=(
            jax.ShapeDtypeStruct((L_LEVELS, Q), jnp.float32),
            jax.ShapeDtypeStruct((L_LEVELS, Q), jnp.float32),
        ),
        mesh=mesh,
        compiler_params=pltpu.CompilerParams(needs_layout_passes=False),
        scratch_types=[
            pltpu.VMEM((chunk,), jnp.float32),
            pltpu.VMEM((chunk,), jnp.float32),
            pltpu.VMEM((T_SIZE,), jnp.float32),
            pltpu.VMEM((T_SIZE,), jnp.float32),
            pltpu.VMEM((chunk,), jnp.float32),
            pltpu.VMEM((chunk,), jnp.float32),
        ],
    )
    def enc(t0_hbm, t1_hbm, px_hbm, py_hbm, o0_hbm, o1_hbm,
            px_v, py_v, tab0_v, tab1_v, a0_v, a1_v):
        wid = lax.axis_index("s") * _NC + lax.axis_index("c")
        base = wid * chunk
        pltpu.sync_copy(px_hbm.at[pl.ds(base, chunk)], px_v)
        pltpu.sync_copy(py_hbm.at[pl.ds(base, chunk)], py_v)
        for l in range(L_LEVELS):
            nl = _NLS[l]
            pltpu.sync_copy(t0_hbm.at[l], tab0_v)
            pltpu.sync_copy(t1_hbm.at[l], tab1_v)
            dense = (nl + 1) ** 2 <= T_SIZE

            def grp(g, _, nl=nl, dense=dense):
                s = pl.ds(g * _VL, _VL)
                x = px_v[s] * jnp.float32(nl)
                y = py_v[s] * jnp.float32(nl)
                ix = x.astype(jnp.int32)
                iy = y.astype(jnp.int32)
                fx = x - ix.astype(jnp.float32)
                fy = y - iy.astype(jnp.float32)
                if dense:
                    b = ix * (nl + 1) + iy
                    i00 = b
                    i01 = b + 1
                    i10 = b + (nl + 1)
                    i11 = b + (nl + 2)
                else:
                    hx0 = ix.astype(jnp.uint32)
                    hx1 = hx0 + jnp.uint32(1)
                    hy0 = iy.astype(jnp.uint32) * _P1
                    hy1 = hy0 + _P1
                    m = jnp.uint32(T_SIZE - 1)
                    i00 = ((hx0 ^ hy0) & m).astype(jnp.int32)
                    i01 = ((hx0 ^ hy1) & m).astype(jnp.int32)
                    i10 = ((hx1 ^ hy0) & m).astype(jnp.int32)
                    i11 = ((hx1 ^ hy1) & m).astype(jnp.int32)
                gx1 = fx
                gx0 = 1.0 - fx
                w00 = gx0 * (1.0 - fy)
                w01 = gx0 * fy
                w10 = gx1 * (1.0 - fy)
                w11 = gx1 * fy
                a0 = (w00 * plsc.load_gather(tab0_v, [i00])
                      + w01 * plsc.load_gather(tab0_v, [i01])
                      + w10 * plsc.load_gather(tab0_v, [i10])
                      + w11 * plsc.load_gather(tab0_v, [i11]))
                a1 = (w00 * plsc.load_gather(tab1_v, [i00])
                      + w01 * plsc.load_gather(tab1_v, [i01])
                      + w10 * plsc.load_gather(tab1_v, [i10])
                      + w11 * plsc.load_gather(tab1_v, [i11]))
                a0_v[s] = a0
                a1_v[s] = a1
                return 0

            lax.fori_loop(0, n_grp, grp, 0)
            pltpu.sync_copy(a0_v, o0_hbm.at[l, pl.ds(base, chunk)])
            pltpu.sync_copy(a1_v, o1_hbm.at[l, pl.ds(base, chunk)])

    return enc(t0, t1, px, py)


def _conv1d(x, W, b):
    y = lax.conv_general_dilated(x, W, window_strides=(2,), padding='SAME',
                                 dimension_numbers=('NWC', 'WIO', 'NWC'))
    return y + b


def _lidar_enc(lidar, W1, b1, W2, b2, W3, b3, ln_s, ln_b):
    lidar = jnp.swapaxes(lidar, -2, -3)
    lidar = lidar.reshape(*lidar.shape[0:-2], -1)
    x = jax.nn.leaky_relu(_conv1d(lidar, W1, b1))
    x = jax.nn.leaky_relu(_conv1d(x, W2, b2))
    x = _conv1d(x, W3, b3)
    x = x.reshape(*x.shape[:-2], -1)
    mu = jnp.mean(x, axis=-1, keepdims=True)
    var = jnp.var(x, axis=-1, keepdims=True)
    x = (x - mu) / jnp.sqrt(var + 1e-6) * ln_s + ln_b
    return jax.nn.leaky_relu(x)


def kernel(self_pos, teammate_positions, opponent_positions,
           opponent_last_known_positions, self_feat, fwd_lidar, rear_lidar,
           teammates, opponents, opponents_last_known, opponent_masks,
           agent_map, unmasked_agent_map, table, fW1, fb1, fW2, fb2, fW3, fb3,
           fln_s, fln_b, rW1, rb1, rW2, rb2, rW3, rb3, rln_s, rln_b, train):
    B = self_pos.shape[0]
    pos_all = jnp.concatenate([
        self_pos.reshape(-1, 2),
        teammate_positions.reshape(-1, 2),
        opponent_positions.reshape(-1, 2),
        opponent_last_known_positions.reshape(-1, 2),
    ], axis=0)
    px = pos_all[:, 0]
    py = pos_all[:, 1]
    t0 = table[:, :, 0]
    t1 = table[:, :, 1]
    o0, o1 = _sc_hash_encode(t0, t1, px, py)
    Q = px.shape[0]
    enc = jnp.stack([o0, o1], axis=-1)
    enc = enc.transpose(1, 0, 2).reshape(Q, 32)
    enc_self = enc[:B]
    enc_tm = enc[B:4 * B].reshape(B, 3, 32)
    enc_op = enc[4 * B:8 * B].reshape(B, 4, 32)
    enc_ol = enc[8 * B:12 * B].reshape(B, 4, 32)
    fwd_l = _lidar_enc(fwd_lidar, fW1, fb1, fW2, fb2, fW3, fb3, fln_s, fln_b)
    rear_l = _lidar_enc(rear_lidar, rW1, rb1, rW2, rb2, rW3, rb3, rln_s, rln_b)
    self_ob = jnp.concatenate([enc_self, self_feat, fwd_l, rear_l], axis=-1)
    tm = jnp.concatenate([enc_tm, teammates], axis=-1)
    op = jnp.concatenate([enc_op, opponents], axis=-1)
    ol = jnp.concatenate([enc_ol, opponents_last_known], axis=-1)
    return (self_ob, tm, op, ol, opponent_masks, agent_map, unmasked_agent_map)
